# baseline (speedup 1.0000x reference)
import jax
import jax.numpy as jnp
from jax.experimental import pallas as pl
from jax.experimental.pallas import tpu as pltpu

_MAIN = 400
_TAIL = 80
_NBUF = 3
_NOBUF = 2


def _chunks(n):
    out = []
    pos = 0
    while n - pos > _MAIN:
        out.append((pos, _MAIN))
        pos += _MAIN
    while pos < n:
        out.append((pos, _TAIL))
        pos += _TAIL
    return out


def _gcn_body(adj_ref, x_ref, w_ref, b_ref, o_ref,
              buf_ref, ostage_ref, insem, outsem):
    n = adj_ref.shape[0]
    chunks = _chunks(n)
    nc = len(chunks)

    def in_copy(c):
        start, rows = chunks[c]
        return pltpu.make_async_copy(
            adj_ref.at[pl.ds(start, rows), :],
            buf_ref.at[c % _NBUF, pl.ds(0, rows), :],
            insem.at[c % _NBUF])

    def out_copy(c):
        start, rows = chunks[c]
        return pltpu.make_async_copy(
            ostage_ref.at[c % _NOBUF, pl.ds(0, rows), :],
            o_ref.at[pl.ds(start, rows), :],
            outsem.at[c % _NOBUF])

    for c in range(_NBUF):
        in_copy(c).start()

    for c, (start, rows) in enumerate(chunks):
        in_copy(c).wait()
        if c >= _NOBUF:
            out_copy(c - _NOBUF).wait()
        ax = jnp.dot(buf_ref[c % _NBUF, 0:rows, :], x_ref[...],
                     preferred_element_type=jnp.float32)
        ostage_ref[c % _NOBUF, 0:rows, :] = jnp.dot(
            ax, w_ref[...], preferred_element_type=jnp.float32) + b_ref[...]
        out_copy(c).start()
        if c + _NBUF < nc:
            in_copy(c + _NBUF).start()

    for c in range(nc - _NOBUF, nc):
        out_copy(c).wait()


def kernel(input, adj, weight, bias):
    n, d_in = input.shape
    d_out = weight.shape[1]

    out = pl.pallas_call(
        _gcn_body,
        in_specs=[
            pl.BlockSpec(memory_space=pltpu.MemorySpace.HBM),
            pl.BlockSpec(memory_space=pltpu.MemorySpace.VMEM),
            pl.BlockSpec(memory_space=pltpu.MemorySpace.VMEM),
            pl.BlockSpec(memory_space=pltpu.MemorySpace.VMEM),
        ],
        out_specs=pl.BlockSpec(memory_space=pltpu.MemorySpace.HBM),
        out_shape=jax.ShapeDtypeStruct((n, d_out), jnp.float32),
        scratch_shapes=[
            pltpu.VMEM((_NBUF, _MAIN, n), jnp.float32),
            pltpu.VMEM((_NOBUF, _MAIN, d_out), jnp.float32),
            pltpu.SemaphoreType.DMA((_NBUF,)),
            pltpu.SemaphoreType.DMA((_NOBUF,)),
        ],
        compiler_params=pltpu.CompilerParams(
            vmem_limit_bytes=62 * 1024 * 1024),
    )(adj, input, weight, bias.reshape(1, d_out))
    return out
